# baseline (speedup 1.0000x reference)
import functools

import jax
import jax.numpy as jnp
from jax import lax
from jax.experimental import pallas as pl
from jax.experimental.pallas import tpu as pltpu
from jax.experimental.pallas import tpu_sc as plsc


def _addend_body(months_ref, ch_ref, pos_ref, mon_ref, out_ref):
    b, t, bs, d = out_ref.shape
    n = ch_ref.shape[1]
    ch = ch_ref[...]
    zero = jnp.zeros((bs, n), jnp.float32)
    for bi in range(b):
        for ti in range(t):
            m = months_ref[bi, ti]
            row_m = mon_ref[pl.ds(m, 1), :]
            row3 = jnp.concatenate([
                ch,
                jnp.broadcast_to(pos_ref[ti:ti + 1, :], (bs, n)),
                jnp.broadcast_to(row_m, (bs, n)),
                zero,
            ], axis=-1)
            out_ref[bi, ti] = row3


def kernel(tokens, timestamps, channel_embed, pos_embed, month_table):
    b, h, w, t, bs, d = tokens.shape
    n = d // 4
    months = timestamps[:, :, 1].astype(jnp.int32)

    a_small = pl.pallas_call(
        _addend_body,
        in_specs=[
            pl.BlockSpec(memory_space=pltpu.SMEM),
            pl.BlockSpec(memory_space=pltpu.VMEM),
            pl.BlockSpec(memory_space=pltpu.VMEM),
            pl.BlockSpec(memory_space=pltpu.VMEM),
        ],
        out_shape=jax.ShapeDtypeStruct((b, t, bs, d), jnp.float32),
    )(months, channel_embed, pos_embed, month_table)

    NC, NS = 2, 16
    NW = NC * NS
    units = b * h * w
    upw = units // NW
    nvec = (3 * n) // 16

    mesh = plsc.VectorSubcoreMesh(core_axis_name="c", subcore_axis_name="s")

    nbuf = 2

    @functools.partial(
        pl.kernel,
        mesh=mesh,
        compiler_params=pltpu.CompilerParams(use_tc_tiling_on_sc=True),
        out_type=jax.ShapeDtypeStruct(tokens.shape, tokens.dtype),
        scratch_types=[
            pltpu.VMEM((nbuf, t, bs, d), jnp.float32),
            pltpu.VMEM((t, bs, d), jnp.float32),
        ]
        + [pltpu.SemaphoreType.DMA] * (2 * nbuf),
    )
    def _sc_add(tok_hbm, a_hbm, out_hbm, buf_v, a_v, *sems):
        sem_in = sems[:nbuf]
        sem_out = sems[nbuf:]
        cid = lax.axis_index("c")
        sid = lax.axis_index("s")
        wid = sid * NC + cid
        b_idx = wid // (NW // b)
        pltpu.sync_copy(a_hbm.at[b_idx], a_v)

        def unit_coords(u):
            bi = u // (h * w)
            rem = u % (h * w)
            return bi, rem // w, rem % w

        def start_in(j, slot):
            bi, hi, wi = unit_coords(wid * upw + j)
            pltpu.make_async_copy(
                tok_hbm.at[bi, hi, wi], buf_v.at[slot], sem_in[slot]).start()

        def start_out(j, slot):
            bi, hi, wi = unit_coords(wid * upw + j)
            pltpu.make_async_copy(
                buf_v.at[slot], out_hbm.at[bi, hi, wi], sem_out[slot]).start()

        def wait_in(j, slot):
            bi, hi, wi = unit_coords(wid * upw + j)
            pltpu.make_async_copy(
                tok_hbm.at[bi, hi, wi], buf_v.at[slot], sem_in[slot]).wait()

        def wait_out(j, slot):
            bi, hi, wi = unit_coords(wid * upw + j)
            pltpu.make_async_copy(
                buf_v.at[slot], out_hbm.at[bi, hi, wi], sem_out[slot]).wait()

        for slot in range(nbuf):
            start_in(slot, slot)

        def outer(j2, carry):
            for slot in range(nbuf):
                j = j2 * nbuf + slot
                wait_in(j, slot)

                def addrow(q, c2):
                    ti = q // bs
                    bsi = q % bs
                    for k in range(nvec):
                        sl = pl.ds(k * 16, 16)
                        buf_v[slot, ti, bsi, sl] = (
                            buf_v[slot, ti, bsi, sl] + a_v[ti, bsi, sl])
                    return c2
                lax.fori_loop(0, t * bs, addrow, 0)
                start_out(j, slot)
                nxt = j + nbuf

                @pl.when(nxt < upw)
                def _():
                    wait_out(nxt - nbuf, slot)
                    start_in(nxt, slot)
            return carry
        lax.fori_loop(0, upw // nbuf, outer, 0)

        for slot in range(nbuf):
            wait_out(upw - nbuf + slot, slot)

    return _sc_add(tokens, a_small)


# speedup vs baseline: 1.2843x; 1.2843x over previous
import jax
import jax.numpy as jnp
from jax import lax
from jax.experimental import pallas as pl
from jax.experimental.pallas import tpu as pltpu


def _addend_body(months_ref, ch_ref, pos_ref, mon_ref, out_ref):
    b, t, bs, d = out_ref.shape
    n = ch_ref.shape[1]
    ch = ch_ref[...]
    zero = jnp.zeros((bs, n), jnp.float32)
    for bi in range(b):
        for ti in range(t):
            m = months_ref[bi, ti]
            row_m = mon_ref[pl.ds(m, 1), :]
            row3 = jnp.concatenate([
                ch,
                jnp.broadcast_to(pos_ref[ti:ti + 1, :], (bs, n)),
                jnp.broadcast_to(row_m, (bs, n)),
                zero,
            ], axis=-1)
            out_ref[bi, ti] = row3


def _make_add_body(b, h, w, t, bs, d):
    lanes = b
    rounds = h

    def _add_body(tok_hbm, a_ref, out_hbm, *scratch):
        bufs = scratch[:2 * lanes]
        sem_in = scratch[2 * lanes:4 * lanes]
        sem_out = scratch[4 * lanes:6 * lanes]

        def start_in(lane, hi, slot):
            pltpu.make_async_copy(
                tok_hbm.at[lane, hi], bufs[lane * 2 + slot],
                sem_in[lane * 2 + slot]).start()

        def wait_in(lane, hi, slot):
            pltpu.make_async_copy(
                tok_hbm.at[lane, hi], bufs[lane * 2 + slot],
                sem_in[lane * 2 + slot]).wait()

        def start_out(lane, hi, slot):
            pltpu.make_async_copy(
                bufs[lane * 2 + slot], out_hbm.at[lane, hi],
                sem_out[lane * 2 + slot]).start()

        def wait_out(lane, hi, slot):
            pltpu.make_async_copy(
                bufs[lane * 2 + slot], out_hbm.at[lane, hi],
                sem_out[lane * 2 + slot]).wait()

        for lane in range(lanes):
            start_in(lane, 0, 0)

        def round_pair(r2, carry):
            for slot in range(2):
                r = r2 * 2 + slot
                for lane in range(lanes):
                    wait_in(lane, r, slot)
                    a_val = a_ref[lane]
                    buf = bufs[lane * 2 + slot]

                    def add_w(k, c2, buf=buf, a_val=a_val):
                        buf[k] = buf[k] + a_val
                        return c2
                    lax.fori_loop(0, w, add_w, 0)
                    start_out(lane, r, slot)

                    @pl.when(jnp.logical_and(r >= 1, r + 1 < rounds))
                    def _(lane=lane, r=r, slot=slot):
                        wait_out(lane, r - 1, 1 - slot)

                    @pl.when(r + 1 < rounds)
                    def _(lane=lane, r=r, slot=slot):
                        start_in(lane, r + 1, 1 - slot)
            return carry
        lax.fori_loop(0, rounds // 2, round_pair, 0)

        for lane in range(lanes):
            wait_out(lane, rounds - 2, 0)
            wait_out(lane, rounds - 1, 1)

    return _add_body


def kernel(tokens, timestamps, channel_embed, pos_embed, month_table):
    b, h, w, t, bs, d = tokens.shape
    months = timestamps[:, :, 1].astype(jnp.int32)

    a_small = pl.pallas_call(
        _addend_body,
        in_specs=[
            pl.BlockSpec(memory_space=pltpu.SMEM),
            pl.BlockSpec(memory_space=pltpu.VMEM),
            pl.BlockSpec(memory_space=pltpu.VMEM),
            pl.BlockSpec(memory_space=pltpu.VMEM),
        ],
        out_shape=jax.ShapeDtypeStruct((b, t, bs, d), jnp.float32),
    )(months, channel_embed, pos_embed, month_table)

    out = pl.pallas_call(
        _make_add_body(b, h, w, t, bs, d),
        in_specs=[
            pl.BlockSpec(memory_space=pl.ANY),
            pl.BlockSpec(memory_space=pltpu.VMEM),
        ],
        out_specs=pl.BlockSpec(memory_space=pl.ANY),
        out_shape=jax.ShapeDtypeStruct(tokens.shape, tokens.dtype),
        scratch_shapes=(
            [pltpu.VMEM((w, t, bs, d), jnp.float32) for _ in range(2 * b)]
            + [pltpu.SemaphoreType.DMA] * (4 * b)
        ),
    )(tokens, a_small)
    return out


# speedup vs baseline: 5.7784x; 4.4994x over previous
import jax
import jax.numpy as jnp
from jax.experimental import pallas as pl
from jax.experimental.pallas import tpu as pltpu


def _addend_body(months_ref, ch_ref, pos_ref, mon_ref, out_ref):
    b, t, bs, d = out_ref.shape
    n = ch_ref.shape[1]
    ch = ch_ref[...]
    zero = jnp.zeros((bs, n), jnp.float32)
    for bi in range(b):
        for ti in range(t):
            m = months_ref[bi, ti]
            row_m = mon_ref[pl.ds(m, 1), :]
            row3 = jnp.concatenate([
                ch,
                jnp.broadcast_to(pos_ref[ti:ti + 1, :], (bs, n)),
                jnp.broadcast_to(row_m, (bs, n)),
                zero,
            ], axis=-1)
            out_ref[bi, ti] = row3


def _add_body(tok_ref, a_ref, out_ref):
    a = a_ref[...]
    out_ref[...] = tok_ref[...] + a[:, None, :, :, None, :]


def kernel(tokens, timestamps, channel_embed, pos_embed, month_table):
    b, h, w, t, bs, d = tokens.shape
    months = timestamps[:, :, 1].astype(jnp.int32)

    a_small = pl.pallas_call(
        _addend_body,
        in_specs=[
            pl.BlockSpec(memory_space=pltpu.SMEM),
            pl.BlockSpec(memory_space=pltpu.VMEM),
            pl.BlockSpec(memory_space=pltpu.VMEM),
            pl.BlockSpec(memory_space=pltpu.VMEM),
        ],
        out_shape=jax.ShapeDtypeStruct((b, t, bs, d), jnp.float32),
    )(months, channel_embed, pos_embed, month_table)

    tok_t = jnp.transpose(tokens, (0, 1, 3, 4, 2, 5))

    H = 2
    out_t = pl.pallas_call(
        _add_body,
        grid=(b, h // H),
        in_specs=[
            pl.BlockSpec((1, H, t, bs, w, d), lambda i, j: (i, j, 0, 0, 0, 0)),
            pl.BlockSpec((1, t, bs, d), lambda i, j: (i, 0, 0, 0)),
        ],
        out_specs=pl.BlockSpec((1, H, t, bs, w, d), lambda i, j: (i, j, 0, 0, 0, 0)),
        out_shape=jax.ShapeDtypeStruct(tok_t.shape, tokens.dtype),
    )(tok_t, a_small)
    return jnp.transpose(out_t, (0, 1, 4, 2, 3, 5))


# speedup vs baseline: 5.9613x; 1.0316x over previous
import jax
import jax.numpy as jnp
from jax.experimental import pallas as pl
from jax.experimental.pallas import tpu as pltpu


def _addend_body(months_ref, ch_ref, pos_ref, mon_ref, out_ref):
    b, t, bs, d = out_ref.shape
    n = ch_ref.shape[1]
    ch = ch_ref[...]
    zero = jnp.zeros((bs, n), jnp.float32)
    for bi in range(b):
        for ti in range(t):
            m = months_ref[bi, ti]
            row_m = mon_ref[pl.ds(m, 1), :]
            row3 = jnp.concatenate([
                ch,
                jnp.broadcast_to(pos_ref[ti:ti + 1, :], (bs, n)),
                jnp.broadcast_to(row_m, (bs, n)),
                zero,
            ], axis=-1)
            out_ref[bi, ti] = row3


def _add_body(tok_ref, a_ref, out_ref):
    a = a_ref[...]
    out_ref[...] = tok_ref[...] + a[:, None, :, :, None, :]


def kernel(tokens, timestamps, channel_embed, pos_embed, month_table):
    b, h, w, t, bs, d = tokens.shape
    months = timestamps[:, :, 1].astype(jnp.int32)

    a_small = pl.pallas_call(
        _addend_body,
        in_specs=[
            pl.BlockSpec(memory_space=pltpu.SMEM),
            pl.BlockSpec(memory_space=pltpu.VMEM),
            pl.BlockSpec(memory_space=pltpu.VMEM),
            pl.BlockSpec(memory_space=pltpu.VMEM),
        ],
        out_shape=jax.ShapeDtypeStruct((b, t, bs, d), jnp.float32),
    )(months, channel_embed, pos_embed, month_table)

    tok_t = jnp.transpose(tokens, (0, 1, 3, 4, 2, 5))

    H = 4
    out_t = pl.pallas_call(
        _add_body,
        grid=(b, h // H),
        in_specs=[
            pl.BlockSpec((1, H, t, bs, w, d), lambda i, j: (i, j, 0, 0, 0, 0)),
            pl.BlockSpec((1, t, bs, d), lambda i, j: (i, 0, 0, 0)),
        ],
        out_specs=pl.BlockSpec((1, H, t, bs, w, d), lambda i, j: (i, j, 0, 0, 0, 0)),
        out_shape=jax.ShapeDtypeStruct(tok_t.shape, tokens.dtype),
    )(tok_t, a_small)
    return jnp.transpose(out_t, (0, 1, 4, 2, 3, 5))
